# baseline (speedup 1.0000x reference)
import functools

import jax
import jax.numpy as jnp
from jax import lax
from jax.experimental import pallas as pl
from jax.experimental.pallas import tpu as pltpu
from jax.experimental.pallas import tpu_sc as plsc

_B = 16384
_D = 64
_NSEM = 5
_NC = 2
_NS = 16
_NW = _NC * _NS
_BPW = _B // _NW
_CHUNK = 128
_NCHUNK = _BPW // _CHUNK
_NIDX = _CHUNK * _NSEM


def _sc_body(data_hbm, w2s_hbm, ivec_hbm, svec_hbm, out_hbm,
             idx_v, idx5_v, sval_v, wv_v, sem_v, outb_v,
             sem_w, sem_s):
    wid = lax.axis_index("s") * _NC + lax.axis_index("c")
    base = wid * _BPW
    pltpu.sync_copy(data_hbm.at[pl.ds(base, _BPW)], idx_v)

    lanes = lax.iota(jnp.int32, 16)
    for c in range(_NCHUNK):
        idx_c = idx_v.at[pl.ds(c * _CHUNK, _CHUNK)]
        cp_w = pltpu.async_copy(ivec_hbm.at[idx_c], wv_v, sem_w)

        for g in range(_CHUNK // 16):
            v5 = idx_v[pl.ds(c * _CHUNK + g * 16, 16)] * _NSEM
            for j in range(_NSEM):
                idx5_v[pl.ds(j * _CHUNK + g * 16, 16)] = v5 + j

        cp_s = pltpu.async_copy(w2s_hbm.at[idx5_v], sval_v, sem_s)
        cp_s.wait()
        cp_g = pltpu.async_copy(svec_hbm.at[sval_v], sem_v, sem_s)
        cp_g.wait()
        cp_w.wait()

        def item(i, carry):
            ss = []
            sq = None
            for q in range(_D // 16):
                sl = pl.ds(q * 16, 16)
                acc = sem_v[i, sl]
                for j in range(1, _NSEM):
                    acc = acc + sem_v[j * _CHUNK + i, sl]
                ss.append(acc)
                sq = acc * acc if sq is None else sq + acc * acc
            total = jnp.maximum(jnp.sum(sq), jnp.float32(1e-24))
            ti = lax.bitcast_convert_type(total, jnp.int32)
            yi = jnp.int32(0x5F3759DF) - lax.shift_right_arithmetic(ti, 1)
            y = lax.bitcast_convert_type(yi, jnp.float32)
            for _ in range(3):
                y = y * (jnp.float32(1.5) - jnp.float32(0.5) * total * y * y)
            half_y = jnp.float32(0.5) * y
            for q in range(_D // 16):
                sl = pl.ds(q * 16, 16)
                outb_v[i, sl] = jnp.float32(0.5) * wv_v[i, sl] + half_y * ss[q]
            return carry

        lax.fori_loop(0, _CHUNK, item, 0)
        pltpu.sync_copy(outb_v, out_hbm.at[pl.ds(base + c * _CHUNK, _CHUNK)])


def kernel(data, widx2sidxs, ivectors, svectors):
    data = data.astype(jnp.int32)
    w2s_flat = widx2sidxs.astype(jnp.int32).reshape(-1)
    mesh = plsc.VectorSubcoreMesh(core_axis_name="c", subcore_axis_name="s")
    f = functools.partial(
        pl.kernel,
        out_type=jax.ShapeDtypeStruct((_B, _D), jnp.float32),
        mesh=mesh,
        compiler_params=pltpu.CompilerParams(use_tc_tiling_on_sc=False,
                                             needs_layout_passes=False),
        scratch_types=[
            pltpu.VMEM((_BPW,), jnp.int32),
            pltpu.VMEM((_NIDX,), jnp.int32),
            pltpu.VMEM((_NIDX,), jnp.int32),
            pltpu.VMEM((_CHUNK, _D), jnp.float32),
            pltpu.VMEM((_NIDX, _D), jnp.float32),
            pltpu.VMEM((_CHUNK, _D), jnp.float32),
            pltpu.SemaphoreType.DMA,
            pltpu.SemaphoreType.DMA,
        ],
    )(_sc_body)
    return f(data, w2s_flat, ivectors, svectors)
